# baseline (speedup 1.0000x reference)
import functools

import jax
import jax.numpy as jnp
from jax import lax
from jax.experimental import pallas as pl
from jax.experimental.pallas import tpu as pltpu
from jax.experimental.pallas import tpu_sc as plsc

_V = 4096
_D = 4
_TM = 1024
_SUB = 256


def _dist_argmin_body(xsq_ref, x2b_ref, cbt_ref, wsq_ref, idx_ref, lsum_ref):
    i = pl.program_id(0)
    mm2 = lax.dot_general(
        x2b_ref[...], cbt_ref[...],
        dimension_numbers=(((1,), (0,)), ((), ())),
        preferred_element_type=jnp.float32,
    )
    xsq = xsq_ref[...]
    wsq = wsq_ref[...]
    nchunk = _V // 128
    nsub = _TM // _SUB
    idx_parts = []
    lsum = jnp.zeros((1, 1), jnp.float32)
    lane = lax.broadcasted_iota(jnp.int32, (_SUB, 128), 1).astype(jnp.float32)
    for s in range(nsub):
        rows = slice(s * _SUB, (s + 1) * _SUB)
        xs = xsq[rows, :]
        m = (xs - mm2[rows, 0:128]) + wsq[:, 0:128]
        cfirst = jnp.zeros((_SUB, 128), jnp.float32)
        for c in range(1, nchunk):
            d = (xs - mm2[rows, c * 128:(c + 1) * 128]) + wsq[:, c * 128:(c + 1) * 128]
            flip = d < m
            m = jnp.where(flip, d, m)
            cfirst = jnp.where(flip, jnp.float32(c), cfirst)

        mtok = jnp.min(m, axis=1, keepdims=True)
        jl = cfirst * 128.0 + lane
        cand = jnp.where(m == mtok, jl, jnp.float32(_V))
        idx_parts.append(jnp.min(cand, axis=1).astype(jnp.int32).reshape(_SUB, 1))
        lsum = lsum + jnp.sum(mtok).reshape(1, 1)
    idx_ref[...] = jnp.concatenate(idx_parts, axis=0)

    @pl.when(i == 0)
    def _():
        lsum_ref[...] = jnp.zeros((1, 1), jnp.float32)

    lsum_ref[...] += lsum


def _dist_argmin(xsq, x2b, cbt, wsq):
    n_tok = x2b.shape[0]
    grid = n_tok // _TM
    return pl.pallas_call(
        _dist_argmin_body,
        grid=(grid,),
        in_specs=[
            pl.BlockSpec((_TM, 1), lambda i: (i, 0)),
            pl.BlockSpec((_TM, _D), lambda i: (i, 0)),
            pl.BlockSpec((_D, _V), lambda i: (0, 0)),
            pl.BlockSpec((1, _V), lambda i: (0, 0)),
        ],
        out_specs=[
            pl.BlockSpec((_TM, 1), lambda i: (i, 0)),
            pl.BlockSpec((1, 1), lambda i: (0, 0)),
        ],
        out_shape=[
            jax.ShapeDtypeStruct((n_tok, 1), jnp.int32),
            jax.ShapeDtypeStruct((1, 1), jnp.float32),
        ],
    )(xsq, x2b, cbt, wsq)


def _sc_gather(table_flat, idx):
    n_tok = idx.shape[0]
    n_words = table_flat.shape[0]
    info = plsc.get_sparse_core_info()
    nw = info.num_cores * info.num_subcores
    b_per_w = n_tok // nw
    mesh = plsc.VectorSubcoreMesh(core_axis_name="c", subcore_axis_name="s")

    @functools.partial(
        pl.kernel, mesh=mesh,
        compiler_params=pltpu.CompilerParams(needs_layout_passes=False),
        out_type=jax.ShapeDtypeStruct((n_tok * _D,), jnp.float32),
        scratch_types=[
            pltpu.VMEM((n_words,), jnp.float32),
            pltpu.VMEM((b_per_w,), jnp.int32),
            pltpu.VMEM((b_per_w * _D,), jnp.float32),
        ],
    )
    def k(table_hbm, idx_hbm, out_hbm, cb_v, idx_v, rows_v):
        wid = lax.axis_index("s") * info.num_cores + lax.axis_index("c")
        base = wid * b_per_w
        pltpu.sync_copy(table_hbm, cb_v)
        pltpu.sync_copy(idx_hbm.at[pl.ds(base, b_per_w)], idx_v)
        lane = lax.broadcasted_iota(jnp.int32, (16,), 0)

        def body(g, _):
            i16 = idx_v[pl.ds(g * 16, 16)]
            w = i16 * _D
            opos = g * (16 * _D) + lane * _D
            for dd in range(_D):
                vals = plsc.load_gather(cb_v, [w + dd])
                plsc.store_scatter(rows_v, [opos + dd], vals)
            return _

        lax.fori_loop(0, b_per_w // 16, body, None)
        pltpu.sync_copy(rows_v, out_hbm.at[pl.ds(base * _D, b_per_w * _D)])

    return k(table_flat, idx)


def kernel(feats, codebook):
    b, l, d = feats.shape
    flat = feats.reshape(-1, d)
    n_tok = flat.shape[0]
    x2b = (2.0 * flat).astype(jnp.bfloat16)
    xsq = jnp.sum(flat ** 2, axis=1, keepdims=True)
    wsq = jnp.sum(codebook ** 2, axis=1).reshape(1, _V)
    cbt = codebook.T

    half = n_tok // 2
    cb_flat = codebook.reshape(-1)
    idx2d_a, lsum_a = _dist_argmin(xsq[:half], x2b[:half], cbt, wsq)
    idx_a = idx2d_a.reshape(half)
    q_a = _sc_gather(cb_flat, idx_a)
    idx2d_b, lsum_b = _dist_argmin(xsq[half:], x2b[half:], cbt, wsq)
    idx_b = idx2d_b.reshape(half)
    q_b = _sc_gather(cb_flat, idx_b)
    idx = jnp.concatenate([idx_a, idx_b])
    lsum = lsum_a + lsum_b

    quant = jnp.concatenate([q_a, q_b]).reshape(b, l, d)
    loss = lsum[0, 0] / jnp.float32(n_tok * d)
    quant_st = feats + lax.stop_gradient(quant - feats)
    return quant_st, idx.reshape(b, l), loss


# speedup vs baseline: 1.0459x; 1.0459x over previous
import functools

import jax
import jax.numpy as jnp
from jax import lax
from jax.experimental import pallas as pl
from jax.experimental.pallas import tpu as pltpu
from jax.experimental.pallas import tpu_sc as plsc

_V = 4096
_D = 4
_TM = 1024
_SUB = 256


def _dist_argmin_body(x_ref, cbt_ref, wsq_ref, idx_ref, lsum_ref):
    i = pl.program_id(0)
    x = x_ref[...]
    x2b = (2.0 * x).astype(jnp.bfloat16)
    xsq = x[:, 0:1] * x[:, 0:1]
    for dd in range(1, _D):
        xsq = xsq + x[:, dd:dd + 1] * x[:, dd:dd + 1]
    mm2 = lax.dot_general(
        x2b, cbt_ref[...],
        dimension_numbers=(((1,), (0,)), ((), ())),
        preferred_element_type=jnp.float32,
    )
    wsq = wsq_ref[...]
    nchunk = _V // 128
    nsub = _TM // _SUB
    idx_parts = []
    lsum = jnp.zeros((1, 1), jnp.float32)
    lane = lax.broadcasted_iota(jnp.int32, (_SUB, 128), 1).astype(jnp.float32)
    for s in range(nsub):
        rows = slice(s * _SUB, (s + 1) * _SUB)
        xs = xsq[rows, :]
        m = (xs - mm2[rows, 0:128]) + wsq[:, 0:128]
        cfirst = jnp.zeros((_SUB, 128), jnp.float32)
        for c in range(1, nchunk):
            d = (xs - mm2[rows, c * 128:(c + 1) * 128]) + wsq[:, c * 128:(c + 1) * 128]
            flip = d < m
            m = jnp.where(flip, d, m)
            cfirst = jnp.where(flip, jnp.float32(c), cfirst)

        mtok = jnp.min(m, axis=1, keepdims=True)
        jl = cfirst * 128.0 + lane
        cand = jnp.where(m == mtok, jl, jnp.float32(_V))
        idx_parts.append(jnp.min(cand, axis=1).astype(jnp.int32).reshape(1, _SUB))
        lsum = lsum + jnp.sum(mtok).reshape(1, 1)
    idx_ref[...] = jnp.concatenate(idx_parts, axis=1).reshape(1, 1, _TM)

    @pl.when(i == 0)
    def _():
        lsum_ref[...] = jnp.zeros((1, 1), jnp.float32)

    lsum_ref[...] += lsum


def _dist_argmin(flat, cbt, wsq):
    n_tok = flat.shape[0]
    grid = n_tok // _TM
    return pl.pallas_call(
        _dist_argmin_body,
        grid=(grid,),
        in_specs=[
            pl.BlockSpec((_TM, _D), lambda i: (i, 0)),
            pl.BlockSpec((_D, _V), lambda i: (0, 0)),
            pl.BlockSpec((1, _V), lambda i: (0, 0)),
        ],
        out_specs=[
            pl.BlockSpec((1, 1, _TM), lambda i: (i, 0, 0)),
            pl.BlockSpec((1, 1), lambda i: (0, 0)),
        ],
        out_shape=[
            jax.ShapeDtypeStruct((grid, 1, _TM), jnp.int32),
            jax.ShapeDtypeStruct((1, 1), jnp.float32),
        ],
    )(flat, cbt, wsq)


def _sc_gather(table_flat, idx):
    n_tok = idx.shape[0]
    n_words = table_flat.shape[0]
    info = plsc.get_sparse_core_info()
    nw = info.num_cores * info.num_subcores
    b_per_w = n_tok // nw
    mesh = plsc.VectorSubcoreMesh(core_axis_name="c", subcore_axis_name="s")

    @functools.partial(
        pl.kernel, mesh=mesh,
        compiler_params=pltpu.CompilerParams(needs_layout_passes=False),
        out_type=jax.ShapeDtypeStruct((n_tok * _D,), jnp.float32),
        scratch_types=[
            pltpu.VMEM((n_words,), jnp.float32),
            pltpu.VMEM((b_per_w,), jnp.int32),
            pltpu.VMEM((b_per_w * _D,), jnp.float32),
        ],
    )
    def k(table_hbm, idx_hbm, out_hbm, cb_v, idx_v, rows_v):
        wid = lax.axis_index("s") * info.num_cores + lax.axis_index("c")
        base = wid * b_per_w
        pltpu.sync_copy(table_hbm, cb_v)
        pltpu.sync_copy(idx_hbm.at[pl.ds(base, b_per_w)], idx_v)
        lane = lax.broadcasted_iota(jnp.int32, (16,), 0)

        def body(g, _):
            i16 = idx_v[pl.ds(g * 16, 16)]
            w = i16 * _D
            opos = g * (16 * _D) + lane * _D
            for dd in range(_D):
                vals = plsc.load_gather(cb_v, [w + dd])
                plsc.store_scatter(rows_v, [opos + dd], vals)
            return _

        lax.fori_loop(0, b_per_w // 16, body, None)
        pltpu.sync_copy(rows_v, out_hbm.at[pl.ds(base * _D, b_per_w * _D)])

    return k(table_flat, idx)


def kernel(feats, codebook):
    b, l, d = feats.shape
    flat = feats.reshape(-1, d)
    n_tok = flat.shape[0]
    wsq = jnp.sum(codebook ** 2, axis=1).reshape(1, _V)
    cbt = codebook.T

    idx3d, lsum = _dist_argmin(flat, cbt, wsq)
    idx = idx3d.reshape(n_tok)

    quant = _sc_gather(codebook.reshape(-1), idx).reshape(b, l, d)
    loss = lsum[0, 0] / jnp.float32(n_tok * d)
    quant_st = feats + lax.stop_gradient(quant - feats)
    return quant_st, idx.reshape(b, l), loss


# speedup vs baseline: 1.0879x; 1.0401x over previous
import functools

import jax
import jax.numpy as jnp
from jax import lax
from jax.experimental import pallas as pl
from jax.experimental.pallas import tpu as pltpu
from jax.experimental.pallas import tpu_sc as plsc

_V = 4096
_D = 4
_TL = 1024
_L = 16384


def _dist_argmin_body(x_ref, cb_ref, idx_ref, lsum_ref):
    bi = pl.program_id(0)
    li = pl.program_id(1)
    x = x_ref[...].reshape(_D, _TL)
    cb = cb_ref[...]
    x2b = (2.0 * x).astype(jnp.bfloat16)
    wsq = cb[:, 0:1] * cb[:, 0:1]
    for dd in range(1, _D):
        wsq = wsq + cb[:, dd:dd + 1] * cb[:, dd:dd + 1]
    xsq = x[0:1, :] * x[0:1, :]
    for dd in range(1, _D):
        xsq = xsq + x[dd:dd + 1, :] * x[dd:dd + 1, :]
    mm2 = lax.dot_general(
        cb, x2b,
        dimension_numbers=(((1,), (0,)), ((), ())),
        preferred_element_type=jnp.float32,
    )
    nchunk = _V // 8
    m = (xsq - mm2[0:8, :]) + wsq[0:8, :]
    cfirst = jnp.zeros((8, _TL), jnp.float32)
    for c in range(1, nchunk):
        d = (xsq - mm2[c * 8:(c + 1) * 8, :]) + wsq[c * 8:(c + 1) * 8, :]
        flip = d < m
        m = jnp.where(flip, d, m)
        cfirst = jnp.where(flip, jnp.float32(c), cfirst)
    mtok = jnp.min(m, axis=0, keepdims=True)
    srow = lax.broadcasted_iota(jnp.int32, (8, _TL), 0).astype(jnp.float32)
    cand = jnp.where(m == mtok, cfirst * 8.0 + srow, jnp.float32(_V))
    idx = jnp.min(cand, axis=0, keepdims=True).astype(jnp.int32)
    idx_ref[...] = idx.reshape(1, 1, _TL)

    @pl.when((bi == 0) & (li == 0))
    def _():
        lsum_ref[...] = jnp.zeros((1, 1), jnp.float32)

    lsum_ref[...] += jnp.sum(mtok).reshape(1, 1)


def _dist_argmin(xt, cb):
    nb = xt.shape[0]
    grid = (nb, _L // _TL)
    return pl.pallas_call(
        _dist_argmin_body,
        grid=grid,
        in_specs=[
            pl.BlockSpec((1, _D, _TL), lambda bi, li: (bi, 0, li)),
            pl.BlockSpec((_V, _D), lambda bi, li: (0, 0)),
        ],
        out_specs=[
            pl.BlockSpec((1, 1, _TL), lambda bi, li: (bi, 0, li)),
            pl.BlockSpec((1, 1), lambda bi, li: (0, 0)),
        ],
        out_shape=[
            jax.ShapeDtypeStruct((nb, 1, _L), jnp.int32),
            jax.ShapeDtypeStruct((1, 1), jnp.float32),
        ],
    )(xt, cb)


def _sc_gather(table_flat, idx):
    n_tok = idx.shape[0]
    n_words = table_flat.shape[0]
    info = plsc.get_sparse_core_info()
    nw = info.num_cores * info.num_subcores
    b_per_w = n_tok // nw
    mesh = plsc.VectorSubcoreMesh(core_axis_name="c", subcore_axis_name="s")

    @functools.partial(
        pl.kernel, mesh=mesh,
        compiler_params=pltpu.CompilerParams(needs_layout_passes=False),
        out_type=jax.ShapeDtypeStruct((n_tok * _D,), jnp.float32),
        scratch_types=[
            pltpu.VMEM((n_words,), jnp.float32),
            pltpu.VMEM((b_per_w,), jnp.int32),
            pltpu.VMEM((b_per_w * _D,), jnp.float32),
        ],
    )
    def k(table_hbm, idx_hbm, out_hbm, cb_v, idx_v, rows_v):
        wid = lax.axis_index("s") * info.num_cores + lax.axis_index("c")
        base = wid * b_per_w
        pltpu.sync_copy(table_hbm, cb_v)
        pltpu.sync_copy(idx_hbm.at[pl.ds(base, b_per_w)], idx_v)

        def body(g, _):
            i16 = idx_v[pl.ds(g * 16, 16)]
            w = i16 * _D
            for dd in range(_D):
                vals = plsc.load_gather(cb_v, [w + dd])
                rows_v[pl.ds(dd * b_per_w + g * 16, 16)] = vals
            return _

        lax.fori_loop(0, b_per_w // 16, body, None)
        bb = base // _L
        l0 = base - bb * _L
        for dd in range(_D):
            pltpu.sync_copy(
                rows_v.at[pl.ds(dd * b_per_w, b_per_w)],
                out_hbm.at[pl.ds((bb * _D + dd) * _L + l0, b_per_w)])

    return k(table_flat, idx)


def kernel(feats, codebook):
    b, l, d = feats.shape
    n_tok = b * l
    xt = feats.transpose(0, 2, 1)

    idx3, lsum = _dist_argmin(xt, codebook)
    idx = idx3.reshape(n_tok)

    q = _sc_gather(codebook.reshape(-1), idx)
    qt = q.reshape(b, d, l)
    qst_t = xt + lax.stop_gradient(qt - xt)
    quant_st = qst_t.transpose(0, 2, 1)
    loss = lsum[0, 0] / jnp.float32(n_tok * d)
    return quant_st, idx3.reshape(b, l), loss


# speedup vs baseline: 1.3415x; 1.2331x over previous
import functools

import jax
import jax.numpy as jnp
from jax import lax
from jax.experimental import pallas as pl
from jax.experimental.pallas import tpu as pltpu
from jax.experimental.pallas import tpu_sc as plsc

_V = 4096
_D = 4
_TM = 1024
_SUB = 256
_L = 16384


def _dist_argmin_body(x_ref, cbt_ref, wsq_ref, idx_ref, lsum_ref):
    i = pl.program_id(0)
    x = x_ref[...]
    x2b = (2.0 * x).astype(jnp.bfloat16)
    xsq = x[:, 0:1] * x[:, 0:1]
    for dd in range(1, _D):
        xsq = xsq + x[:, dd:dd + 1] * x[:, dd:dd + 1]
    mm2 = lax.dot_general(
        x2b, cbt_ref[...],
        dimension_numbers=(((1,), (0,)), ((), ())),
        preferred_element_type=jnp.float32,
    )
    wsq = wsq_ref[...]
    nchunk = _V // 128
    nsub = _TM // _SUB
    idx_parts = []
    lsum = jnp.zeros((1, 1), jnp.float32)
    lane = lax.broadcasted_iota(jnp.int32, (_SUB, 128), 1).astype(jnp.float32)
    for s in range(nsub):
        rows = slice(s * _SUB, (s + 1) * _SUB)
        xs = xsq[rows, :]
        m = (xs - mm2[rows, 0:128]) + wsq[:, 0:128]
        cfirst = jnp.zeros((_SUB, 128), jnp.float32)
        for c in range(1, nchunk):
            d = (xs - mm2[rows, c * 128:(c + 1) * 128]) + wsq[:, c * 128:(c + 1) * 128]
            flip = d < m
            m = jnp.where(flip, d, m)
            cfirst = jnp.where(flip, jnp.float32(c), cfirst)

        mtok = jnp.min(m, axis=1, keepdims=True)
        jl = cfirst * 128.0 + lane
        cand = jnp.where(m == mtok, jl, jnp.float32(_V))
        idx_parts.append(jnp.min(cand, axis=1).astype(jnp.int32).reshape(1, _SUB))
        lsum = lsum + jnp.sum(mtok).reshape(1, 1)
    idx_ref[...] = jnp.concatenate(idx_parts, axis=1).reshape(1, 1, _TM)

    @pl.when(i == 0)
    def _():
        lsum_ref[...] = jnp.zeros((1, 1), jnp.float32)

    lsum_ref[...] += lsum


def _dist_argmin(flat, cbt, wsq):
    n_tok = flat.shape[0]
    grid = n_tok // _TM
    return pl.pallas_call(
        _dist_argmin_body,
        grid=(grid,),
        in_specs=[
            pl.BlockSpec((_TM, _D), lambda i: (i, 0)),
            pl.BlockSpec((_D, _V), lambda i: (0, 0)),
            pl.BlockSpec((1, _V), lambda i: (0, 0)),
        ],
        out_specs=[
            pl.BlockSpec((1, 1, _TM), lambda i: (i, 0, 0)),
            pl.BlockSpec((1, 1), lambda i: (0, 0)),
        ],
        out_shape=[
            jax.ShapeDtypeStruct((grid, 1, _TM), jnp.int32),
            jax.ShapeDtypeStruct((1, 1), jnp.float32),
        ],
    )(flat, cbt, wsq)


def _sc_gather(table_flat, idx):
    n_tok = idx.shape[0]
    n_words = table_flat.shape[0]
    info = plsc.get_sparse_core_info()
    nw = info.num_cores * info.num_subcores
    b_per_w = n_tok // nw
    mesh = plsc.VectorSubcoreMesh(core_axis_name="c", subcore_axis_name="s")

    @functools.partial(
        pl.kernel, mesh=mesh,
        compiler_params=pltpu.CompilerParams(needs_layout_passes=False),
        out_type=jax.ShapeDtypeStruct((n_tok * _D,), jnp.float32),
        scratch_types=[
            pltpu.VMEM((n_words,), jnp.float32),
            pltpu.VMEM((b_per_w,), jnp.int32),
            pltpu.VMEM((b_per_w * _D,), jnp.float32),
        ],
    )
    def k(table_hbm, idx_hbm, out_hbm, cb_v, idx_v, rows_v):
        wid = lax.axis_index("s") * info.num_cores + lax.axis_index("c")
        base = wid * b_per_w
        pltpu.sync_copy(table_hbm, cb_v)
        pltpu.sync_copy(idx_hbm.at[pl.ds(base, b_per_w)], idx_v)

        def body(g, _):
            i16 = idx_v[pl.ds(g * 16, 16)]
            w = i16 * _D
            for dd in range(_D):
                vals = plsc.load_gather(cb_v, [w + dd])
                rows_v[pl.ds(dd * b_per_w + g * 16, 16)] = vals
            return _

        lax.fori_loop(0, b_per_w // 16, body, None)
        bb = base // _L
        l0 = base - bb * _L
        for dd in range(_D):
            pltpu.sync_copy(
                rows_v.at[pl.ds(dd * b_per_w, b_per_w)],
                out_hbm.at[pl.ds((bb * _D + dd) * _L + l0, b_per_w)])

    return k(table_flat, idx)


def kernel(feats, codebook):
    b, l, d = feats.shape
    flat = feats.reshape(-1, d)
    n_tok = flat.shape[0]
    wsq = jnp.sum(codebook ** 2, axis=1).reshape(1, _V)
    cbt = codebook.T

    idx3, lsum = _dist_argmin(flat, cbt, wsq)
    idx = idx3.reshape(n_tok)

    q = _sc_gather(codebook.reshape(-1), idx)
    qt = q.reshape(b, d, l)
    xt = feats.transpose(0, 2, 1)
    qst_t = xt + lax.stop_gradient(qt - xt)
    quant_st = qst_t.transpose(0, 2, 1)
    loss = lsum[0, 0] / jnp.float32(n_tok * d)
    return quant_st, idx3.reshape(b, l), loss


# speedup vs baseline: 1.4006x; 1.0441x over previous
import functools

import jax
import jax.numpy as jnp
from jax import lax
from jax.experimental import pallas as pl
from jax.experimental.pallas import tpu as pltpu
from jax.experimental.pallas import tpu_sc as plsc

_V = 4096
_D = 4
_TM = 1024
_SUB = 256
_L = 16384


def _dist_argmin_body(x_ref, cbt_ref, wsq_ref, idx_ref, lsum_ref):
    i = pl.program_id(0)
    x = x_ref[...]
    x2b = (2.0 * x).astype(jnp.bfloat16)
    xsq = (x[:, 0:1] * x[:, 0:1] + x[:, 1:2] * x[:, 1:2]) + (
        x[:, 2:3] * x[:, 2:3] + x[:, 3:4] * x[:, 3:4])
    mm2 = lax.dot_general(
        x2b, cbt_ref[...],
        dimension_numbers=(((1,), (0,)), ((), ())),
        preferred_element_type=jnp.float32,
    )
    wsq = wsq_ref[...]
    nchunk = _V // 128
    nsub = _TM // _SUB
    idx_parts = []
    lsum = jnp.zeros((1, 1), jnp.float32)
    lane = lax.broadcasted_iota(jnp.int32, (_SUB, 128), 1).astype(jnp.float32)
    for s in range(nsub):
        rows = slice(s * _SUB, (s + 1) * _SUB)
        xs = xsq[rows, :]
        m = (xs - mm2[rows, 0:128]) + wsq[:, 0:128]
        cfirst = jnp.zeros((_SUB, 128), jnp.float32)
        for c in range(1, nchunk):
            d = (xs - mm2[rows, c * 128:(c + 1) * 128]) + wsq[:, c * 128:(c + 1) * 128]
            flip = d < m
            m = jnp.where(flip, d, m)
            cfirst = jnp.where(flip, jnp.float32(c), cfirst)

        mtok = jnp.min(m, axis=1, keepdims=True)
        jl = cfirst * 128.0 + lane
        cand = jnp.where(m == mtok, jl, jnp.float32(_V))
        idx_parts.append(jnp.min(cand, axis=1).astype(jnp.int32).reshape(1, _SUB))
        lsum = lsum + jnp.sum(mtok).reshape(1, 1)
    idx_ref[...] = jnp.concatenate(idx_parts, axis=1).reshape(1, 1, _TM)

    @pl.when(i == 0)
    def _():
        lsum_ref[...] = jnp.zeros((1, 1), jnp.float32)

    lsum_ref[...] += lsum


def _dist_argmin(flat, cbt, wsq):
    n_tok = flat.shape[0]
    grid = n_tok // _TM
    return pl.pallas_call(
        _dist_argmin_body,
        grid=(grid,),
        in_specs=[
            pl.BlockSpec((_TM, _D), lambda i: (i, 0)),
            pl.BlockSpec((_D, _V), lambda i: (0, 0)),
            pl.BlockSpec((1, _V), lambda i: (0, 0)),
        ],
        out_specs=[
            pl.BlockSpec((1, 1, _TM), lambda i: (i, 0, 0)),
            pl.BlockSpec((1, 1), lambda i: (0, 0)),
        ],
        out_shape=[
            jax.ShapeDtypeStruct((grid, 1, _TM), jnp.int32),
            jax.ShapeDtypeStruct((1, 1), jnp.float32),
        ],
    )(flat, cbt, wsq)


def _sc_gather(table_flat, idx):
    n_tok = idx.shape[0]
    n_words = table_flat.shape[0]
    info = plsc.get_sparse_core_info()
    nw = info.num_cores * info.num_subcores
    b_per_w = n_tok // nw
    mesh = plsc.VectorSubcoreMesh(core_axis_name="c", subcore_axis_name="s")

    @functools.partial(
        pl.kernel, mesh=mesh,
        compiler_params=pltpu.CompilerParams(needs_layout_passes=False),
        out_type=jax.ShapeDtypeStruct((n_tok * _D,), jnp.float32),
        scratch_types=[
            pltpu.VMEM((n_words,), jnp.float32),
            pltpu.VMEM((b_per_w,), jnp.int32),
            pltpu.VMEM((b_per_w * _D,), jnp.float32),
        ],
    )
    def k(table_hbm, idx_hbm, out_hbm, cb_v, idx_v, rows_v):
        wid = lax.axis_index("s") * info.num_cores + lax.axis_index("c")
        base = wid * b_per_w
        pltpu.sync_copy(table_hbm, cb_v)
        pltpu.sync_copy(idx_hbm.at[pl.ds(base, b_per_w)], idx_v)

        def body(g, _):
            i16 = idx_v[pl.ds(g * 16, 16)]
            w = i16 * _D
            for dd in range(_D):
                vals = plsc.load_gather(cb_v, [w + dd])
                rows_v[pl.ds(dd * b_per_w + g * 16, 16)] = vals
            return _

        lax.fori_loop(0, b_per_w // 16, body, None)
        bb = base // _L
        l0 = base - bb * _L
        for dd in range(_D):
            pltpu.sync_copy(
                rows_v.at[pl.ds(dd * b_per_w, b_per_w)],
                out_hbm.at[pl.ds((bb * _D + dd) * _L + l0, b_per_w)])

    return k(table_flat, idx)


def kernel(feats, codebook):
    b, l, d = feats.shape
    flat = feats.reshape(-1, d)
    n_tok = flat.shape[0]
    wsq = jnp.sum(codebook ** 2, axis=1).reshape(1, _V)
    cbt = codebook.T

    idx3, lsum = _dist_argmin(flat, cbt, wsq)
    idx = idx3.reshape(n_tok)

    q = _sc_gather(codebook.reshape(-1), idx)
    qt = q.reshape(b, d, l)
    xt = feats.transpose(0, 2, 1)
    qst_t = xt + lax.stop_gradient(qt - xt)
    quant_st = qst_t.transpose(0, 2, 1)
    loss = lsum[0, 0] / jnp.float32(n_tok * d)
    return quant_st, idx3.reshape(b, l), loss
